# Initial kernel scaffold; baseline (speedup 1.0000x reference)
import jax
import jax.numpy as jnp
from jax.experimental import pallas as pl


def kernel(x, pos, batch, y, params):
    raise NotImplementedError("write your pallas kernel here")


# speedup vs baseline: 1.0001x; 1.0001x over previous
import jax
import jax.numpy as jnp
from jax.experimental import pallas as pl

_EPS = 1e-5
_MAX_NN = 64


def _bn_stack(layers, h, mask=None):
    for (W, b, g, bt) in layers:
        h = jnp.maximum(h @ W + b, 0.0)
        if mask is None:
            mean = jnp.mean(h, axis=0)
            var = jnp.mean((h - mean) ** 2, axis=0)
        else:
            m = mask[:, None].astype(h.dtype)
            cnt = jnp.maximum(jnp.sum(m), 1.0)
            mean = jnp.sum(h * m, axis=0) / cnt
            var = jnp.sum(((h - mean) ** 2) * m, axis=0) / cnt
        h = (h - mean) * jax.lax.rsqrt(var + _EPS) * g + bt
    return h


def _fps_v0(pos, n_samples):
    pos = jax.lax.stop_gradient(pos)
    N = pos.shape[0]
    idxs = jnp.zeros((n_samples,), dtype=jnp.int32)
    dists = jnp.full((N,), jnp.inf, dtype=jnp.float32)

    def body(i, carry):
        dists, idxs = carry
        last = idxs[i - 1]
        d = jnp.sum((pos - pos[last]) ** 2, axis=1)
        dists = jnp.minimum(dists, d)
        nxt = jnp.argmax(dists).astype(jnp.int32)
        idxs = idxs.at[i].set(nxt)
        return (dists, idxs)

    _, idxs = jax.lax.fori_loop(1, n_samples, body, (dists, idxs))
    return idxs


def _radius_v0(pos_x, pos_y, r, max_nn):
    px = jax.lax.stop_gradient(pos_x)
    py = jax.lax.stop_gradient(pos_y)
    d2 = jnp.sum(py ** 2, axis=1)[:, None] + jnp.sum(px ** 2, axis=1)[None, :] - 2.0 * (py @ px.T)
    valid = d2 <= r * r
    Nx = px.shape[0]
    Ny = py.shape[0]
    score = jnp.where(valid, -jnp.arange(Nx, dtype=jnp.float32)[None, :], -jnp.inf)
    top_s, top_i = jax.lax.top_k(score, max_nn)
    row = jnp.repeat(jnp.arange(Ny, dtype=jnp.int32), max_nn)
    col = top_i.reshape(-1).astype(jnp.int32)
    mask = (top_s > -jnp.inf).reshape(-1)
    return row, col, mask


def _sa_v0(x, pos, ratio, r, layers):
    N = pos.shape[0]
    n_s = int(N * ratio)
    idx = _fps_v0(pos, n_s)
    pos_y = pos[idx]
    row, col, mask = _radius_v0(pos, pos_y, r, _MAX_NN)
    msg_in = jnp.concatenate([x[col], pos[col] - pos_y[row]], axis=1)
    msg = _bn_stack(layers, msg_in, mask=mask)
    msg = jnp.where(mask[:, None], msg, -jnp.inf)
    out = jax.ops.segment_max(msg, row, num_segments=n_s)
    out = jnp.where(jnp.isfinite(out), out, 0.0)
    return out, pos_y


def _head_kernel(h_ref, g_ref, w1_ref, b1_ref, w2_ref, b2_ref, w3_ref, b3_ref, o_ref):
    h = jnp.max(h_ref[...], axis=0, keepdims=True)
    hg = jnp.concatenate([h, g_ref[...]], axis=1)
    o = jnp.maximum(hg @ w1_ref[...] + b1_ref[...], 0.0)
    o = jnp.maximum(o @ w2_ref[...] + b2_ref[...], 0.0)
    o_ref[...] = o @ w3_ref[...] + b3_ref[...]


def kernel(x, pos, batch, y, params):
    N = x.shape[0]
    perm = jax.random.permutation(jax.random.key(42), N)
    x = x[perm]
    pos = pos[perm]
    x1, pos1 = _sa_v0(x, pos, 0.5, 0.2, params['sa1'])
    x1 = x1[: N // 4]
    pos1 = pos1[: N // 4]
    x2, pos2 = _sa_v0(x1, pos1, 0.25, 0.4, params['sa2'])
    x2 = x2[: N // 16]
    pos2 = pos2[: N // 16]
    h = _bn_stack(params['sa3'], jnp.concatenate([x2, pos2], axis=1))
    g = y[:, 1:5].reshape(-1, 4)
    W1, b1 = params['lin1']
    W2, b2 = params['lin2']
    W3, b3 = params['lin3']
    out = pl.pallas_call(
        _head_kernel,
        out_shape=jax.ShapeDtypeStruct((1, 1), jnp.float32),
    )(h, g, W1, b1[None, :], W2, b2[None, :], W3, b3[None, :])
    return out.reshape(-1)


# speedup vs baseline: 2.6506x; 2.6503x over previous
import functools

import jax
import jax.numpy as jnp
from jax.experimental import pallas as pl

_EPS = 1e-5
_MAX_NN = 64


def _fps_body(posT_ref, posN_ref, pos_y_ref, *, n_s, N, coff):
    R = N // 128
    x0 = posT_ref[0]
    x1 = posT_ref[1]
    x2 = posT_ref[2]
    iota = (jax.lax.broadcasted_iota(jnp.int32, (R, 128), 0) * 128
            + jax.lax.broadcasted_iota(jnp.int32, (R, 128), 1))

    def body(i, carry):
        dists, nxt_prev = carry
        p_row = posN_ref[pl.ds(nxt_prev, 1), :]
        pos_y_ref[pl.ds(i - 1, 1), :] = p_row
        p0 = p_row[0, coff]
        p1 = p_row[0, coff + 1]
        p2 = p_row[0, coff + 2]
        d = (x0 - p0) ** 2 + (x1 - p1) ** 2 + (x2 - p2) ** 2
        dists = jnp.minimum(dists, d)
        m = jnp.max(dists)
        nxt = jnp.min(jnp.where(dists == m, iota, N))
        return (dists, nxt)

    dists0 = jnp.full((R, 128), jnp.inf, dtype=jnp.float32)
    _, nxt_last = jax.lax.fori_loop(1, n_s, body, (dists0, jnp.int32(0)))
    pos_y_ref[pl.ds(n_s - 1, 1), :] = posN_ref[pl.ds(nxt_last, 1), :]


def _fps_pallas(posT, posN, n_s, coff):
    N, W = posN.shape
    return pl.pallas_call(
        functools.partial(_fps_body, n_s=n_s, N=N, coff=coff),
        out_shape=jax.ShapeDtypeStruct((n_s, W), jnp.float32),
    )(posT, posN)


def _bn_stack(layers, h, mask=None):
    for (W, b, g, bt) in layers:
        h = jnp.maximum(h @ W + b, 0.0)
        if mask is None:
            mean = jnp.mean(h, axis=0)
            var = jnp.mean((h - mean) ** 2, axis=0)
        else:
            m = mask[:, None].astype(h.dtype)
            cnt = jnp.maximum(jnp.sum(m), 1.0)
            mean = jnp.sum(h * m, axis=0) / cnt
            var = jnp.sum(((h - mean) ** 2) * m, axis=0) / cnt
        h = (h - mean) * jax.lax.rsqrt(var + _EPS) * g + bt
    return h


def _fps_v0(pos, n_samples):
    pos = jax.lax.stop_gradient(pos)
    N = pos.shape[0]
    idxs = jnp.zeros((n_samples,), dtype=jnp.int32)
    dists = jnp.full((N,), jnp.inf, dtype=jnp.float32)

    def body(i, carry):
        dists, idxs = carry
        last = idxs[i - 1]
        d = jnp.sum((pos - pos[last]) ** 2, axis=1)
        dists = jnp.minimum(dists, d)
        nxt = jnp.argmax(dists).astype(jnp.int32)
        idxs = idxs.at[i].set(nxt)
        return (dists, idxs)

    _, idxs = jax.lax.fori_loop(1, n_samples, body, (dists, idxs))
    return idxs


def _radius_v0(pos_x, pos_y, r, max_nn):
    px = jax.lax.stop_gradient(pos_x)
    py = jax.lax.stop_gradient(pos_y)
    d2 = jnp.sum(py ** 2, axis=1)[:, None] + jnp.sum(px ** 2, axis=1)[None, :] - 2.0 * (py @ px.T)
    valid = d2 <= r * r
    Nx = px.shape[0]
    Ny = py.shape[0]
    score = jnp.where(valid, -jnp.arange(Nx, dtype=jnp.float32)[None, :], -jnp.inf)
    top_s, top_i = jax.lax.top_k(score, max_nn)
    row = jnp.repeat(jnp.arange(Ny, dtype=jnp.int32), max_nn)
    col = top_i.reshape(-1).astype(jnp.int32)
    mask = (top_s > -jnp.inf).reshape(-1)
    return row, col, mask


def _sa_v0(x, pos, ratio, r, layers):
    N = pos.shape[0]
    n_s = int(N * ratio)
    posT = pos.T.reshape(3, N // 128, 128)
    posN = jnp.concatenate([pos, pos, jnp.zeros((N, 10), jnp.float32)], axis=1)
    pos_y = _fps_pallas(posT, posN, n_s, 0)[:, 0:3]
    row, col, mask = _radius_v0(pos, pos_y, r, _MAX_NN)
    msg_in = jnp.concatenate([x[col], pos[col] - pos_y[row]], axis=1)
    msg = _bn_stack(layers, msg_in, mask=mask)
    msg = jnp.where(mask[:, None], msg, -jnp.inf)
    out = jax.ops.segment_max(msg, row, num_segments=n_s)
    out = jnp.where(jnp.isfinite(out), out, 0.0)
    return out, pos_y


def _dot(a, b):
    return jax.lax.dot_general(a, b, (((a.ndim - 1,), (0,)), ((), ())),
                               precision=jax.lax.Precision.HIGHEST)


def _head_kernel(h_ref, g_ref, w1_ref, b1_ref, w2_ref, b2_ref, w3_ref, b3_ref, o_ref):
    h = jnp.max(h_ref[...], axis=0, keepdims=True)
    hg = jnp.concatenate([h, g_ref[...]], axis=1)
    o = jnp.maximum(_dot(hg, w1_ref[...]) + b1_ref[...], 0.0)
    o = jnp.maximum(_dot(o, w2_ref[...]) + b2_ref[...], 0.0)
    o_ref[...] = _dot(o, w3_ref[...]) + b3_ref[...]


def kernel(x, pos, batch, y, params):
    N = x.shape[0]
    perm = jax.random.permutation(jax.random.key(42), N)
    x = x[perm]
    pos = pos[perm]
    x1, pos1 = _sa_v0(x, pos, 0.5, 0.2, params['sa1'])
    x1 = x1[: N // 4]
    pos1 = pos1[: N // 4]
    x2, pos2 = _sa_v0(x1, pos1, 0.25, 0.4, params['sa2'])
    x2 = x2[: N // 16]
    pos2 = pos2[: N // 16]
    h = _bn_stack(params['sa3'], jnp.concatenate([x2, pos2], axis=1))
    g = y[:, 1:5].reshape(-1, 4)
    W1, b1 = params['lin1']
    W2, b2 = params['lin2']
    W3, b3 = params['lin3']
    out = pl.pallas_call(
        _head_kernel,
        out_shape=jax.ShapeDtypeStruct((1, 1), jnp.float32),
    )(h, g, W1, b1[None, :], W2, b2[None, :], W3, b3[None, :])
    return out.reshape(-1)


# speedup vs baseline: 2.6518x; 1.0005x over previous
import functools

import jax
import jax.numpy as jnp
from jax.experimental import pallas as pl

_EPS = 1e-5
_MAX_NN = 64


def _fps_body(posT_ref, posN_ref, pos_y_ref, *, n_s, N, coff):
    R = N // 128
    x0 = posT_ref[0]
    x1 = posT_ref[1]
    x2 = posT_ref[2]
    iota = (jax.lax.broadcasted_iota(jnp.int32, (R, 128), 0) * 128
            + jax.lax.broadcasted_iota(jnp.int32, (R, 128), 1))

    def body(i, carry):
        dists, nxt_prev = carry
        p_row = posN_ref[pl.ds(nxt_prev, 1), :]
        pos_y_ref[pl.ds(i - 1, 1), :] = p_row
        p0 = p_row[0, coff]
        p1 = p_row[0, coff + 1]
        p2 = p_row[0, coff + 2]
        d = (x0 - p0) ** 2 + (x1 - p1) ** 2 + (x2 - p2) ** 2
        dists = jnp.minimum(dists, d)
        m = jnp.max(dists)
        nxt = jnp.min(jnp.where(dists == m, iota, N))
        return (dists, nxt)

    dists0 = jnp.full((R, 128), jnp.inf, dtype=jnp.float32)
    _, nxt_last = jax.lax.fori_loop(1, n_s, body, (dists0, jnp.int32(0)))
    pos_y_ref[pl.ds(n_s - 1, 1), :] = posN_ref[pl.ds(nxt_last, 1), :]


def _fps_pallas(posT, posN, n_s, coff):
    N, W = posN.shape
    return pl.pallas_call(
        functools.partial(_fps_body, n_s=n_s, N=N, coff=coff),
        out_shape=jax.ShapeDtypeStruct((n_s, W), jnp.float32),
    )(posT, posN)


def _bn_stack(layers, h, mask=None):
    for (W, b, g, bt) in layers:
        h = jnp.maximum(h @ W + b, 0.0)
        if mask is None:
            mean = jnp.mean(h, axis=0)
            var = jnp.mean((h - mean) ** 2, axis=0)
        else:
            m = mask[:, None].astype(h.dtype)
            cnt = jnp.maximum(jnp.sum(m), 1.0)
            mean = jnp.sum(h * m, axis=0) / cnt
            var = jnp.sum(((h - mean) ** 2) * m, axis=0) / cnt
        h = (h - mean) * jax.lax.rsqrt(var + _EPS) * g + bt
    return h


def _radius_v0(pos_x, pos_y, r, max_nn):
    px = jax.lax.stop_gradient(pos_x)
    py = jax.lax.stop_gradient(pos_y)
    d2 = (jnp.sum(py ** 2, axis=1)[:, None] + jnp.sum(px ** 2, axis=1)[None, :]
          - 2.0 * (py @ px.T))
    valid = d2 <= r * r
    Nx = px.shape[0]
    Ny = py.shape[0]
    score = jnp.where(valid, -jnp.arange(Nx, dtype=jnp.float32)[None, :], -jnp.inf)
    top_s, top_i = jax.lax.top_k(score, max_nn)
    row = jnp.repeat(jnp.arange(Ny, dtype=jnp.int32), max_nn)
    col = top_i.reshape(-1).astype(jnp.int32)
    mask = (top_s > -jnp.inf).reshape(-1)
    return row, col, mask


def _fps_ref(pos, n_samples):
    pos = jax.lax.stop_gradient(pos)
    N = pos.shape[0]
    idxs = jnp.zeros((n_samples,), dtype=jnp.int32)
    dists = jnp.full((N,), jnp.inf, dtype=jnp.float32)

    def body(i, carry):
        dists, idxs = carry
        last = idxs[i - 1]
        d = jnp.sum((pos - pos[last]) ** 2, axis=1)
        dists = jnp.minimum(dists, d)
        nxt = jnp.argmax(dists).astype(jnp.int32)
        idxs = idxs.at[i].set(nxt)
        return (dists, idxs)

    _, idxs = jax.lax.fori_loop(1, n_samples, body, (dists, idxs))
    return idxs


def _sa_v0(x, pos, ratio, r, layers):
    N = pos.shape[0]
    n_s = int(N * ratio)
    posT = pos.T.reshape(3, N // 128, 128)
    posN = jnp.concatenate([pos, pos, jnp.zeros((N, 10), jnp.float32)], axis=1)
    pos_y = _fps_pallas(posT, posN, n_s, 0)[:, 0:3]
    row, col, mask = _radius_v0(pos, pos_y, r, _MAX_NN)
    msg_in = jnp.concatenate([x[col], pos[col] - pos_y[row]], axis=1)
    msg = _bn_stack(layers, msg_in, mask=mask)
    msg = jnp.where(mask[:, None], msg, -jnp.inf)
    out = jax.ops.segment_max(msg, row, num_segments=n_s)
    out = jnp.where(jnp.isfinite(out), out, 0.0)
    return out, pos_y


def _dot(a, b):
    return jax.lax.dot_general(a, b, (((a.ndim - 1,), (0,)), ((), ())),
                               precision=jax.lax.Precision.HIGHEST)


def _head_kernel(h_ref, g_ref, w1_ref, b1_ref, w2_ref, b2_ref, w3_ref, b3_ref, o_ref):
    h = jnp.max(h_ref[...], axis=0, keepdims=True)
    hg = jnp.concatenate([h, g_ref[...]], axis=1)
    o = jnp.maximum(_dot(hg, w1_ref[...]) + b1_ref[...], 0.0)
    o = jnp.maximum(_dot(o, w2_ref[...]) + b2_ref[...], 0.0)
    o_ref[...] = _dot(o, w3_ref[...]) + b3_ref[...]


def kernel(x, pos, batch, y, params):
    N = x.shape[0]
    perm = jax.random.permutation(jax.random.key(42), N)
    x = x[perm]
    pos = pos[perm]
    x1, pos1 = _sa_v0(x, pos, 0.5, 0.2, params['sa1'])
    x1 = x1[: N // 4]
    pos1 = pos1[: N // 4]
    x2, pos2 = _sa_v0(x1, pos1, 0.25, 0.4, params['sa2'])
    x2 = x2[: N // 16]
    pos2 = pos2[: N // 16]
    h = _bn_stack(params['sa3'], jnp.concatenate([x2, pos2], axis=1))
    out = jnp.max(h, axis=0, keepdims=True)
    out = out[: N // 64]
    g = y[:, 1:5].reshape(-1, 4)
    out = jnp.concatenate([out, g], axis=1)
    W1, b1 = params['lin1']
    out = jnp.maximum(out @ W1 + b1, 0.0)
    W2, b2 = params['lin2']
    out = jnp.maximum(out @ W2 + b2, 0.0)
    W3, b3 = params['lin3']
    out = out @ W3 + b3
    return out.reshape(-1)
